# baseline (speedup 1.0000x reference)
import functools

import jax
import jax.numpy as jnp
from jax.experimental import pallas as pl

KNN_K = 32
TEMP = 0.1


def _mm_kernel(n_valid, x_ref, mu_ref, o_ref):
    j = pl.program_id(1)
    s = jax.lax.dot_general(
        x_ref[...], mu_ref[...],
        (((1,), (1,)), ((), ())),
        preferred_element_type=jnp.float32,
    )
    col = jax.lax.broadcasted_iota(jnp.int32, s.shape, 1) + j * s.shape[1]
    o_ref[...] = jnp.where(col < n_valid, s, -2.0)


def _geom_kernel(x_ref, xt_ref, o_ref):
    i = pl.program_id(0)
    j = pl.program_id(1)
    s = jax.lax.dot_general(
        x_ref[...], xt_ref[...],
        (((1,), (1,)), ((), ())),
        preferred_element_type=jnp.float32,
    )
    rb, cb = s.shape
    row = jax.lax.broadcasted_iota(jnp.int32, s.shape, 0) + i * rb
    col = jax.lax.broadcasted_iota(jnp.int32, s.shape, 1) + j * cb
    vals = -jnp.log(1.0 - s + 0.0001)
    vals = jnp.where(row == col, 0.0, vals)

    @pl.when((i == 0) & (j == 0))
    def _():
        o_ref[...] = jnp.zeros_like(o_ref)

    o_ref[...] += jnp.sum(vals)[None, None]


def kernel(x, mu, alpha, W_comp_w, W_comp_b):
    B, D = x.shape
    N = mu.shape[0]
    CN = 2048
    NP = ((N + CN - 1) // CN) * CN
    RB = 2048
    mu_p = jnp.pad(mu, ((0, NP - N), (0, 0)))

    sims = pl.pallas_call(
        functools.partial(_mm_kernel, N),
        grid=(B // RB, NP // CN),
        in_specs=[
            pl.BlockSpec((RB, D), lambda i, j: (i, 0)),
            pl.BlockSpec((CN, D), lambda i, j: (j, 0)),
        ],
        out_specs=pl.BlockSpec((RB, CN), lambda i, j: (i, j)),
        out_shape=jax.ShapeDtypeStruct((B, NP), jnp.float32),
    )(x, mu_p)

    vals, idx = jax.lax.top_k(sims, KNN_K)
    n_alpha = jnp.take(alpha, idx, axis=0)
    exponent = n_alpha * (vals - 1.0) / TEMP
    e_splat = -jax.nn.logsumexp(exponent, axis=-1)

    GB = 1024
    geom_parts = pl.pallas_call(
        _geom_kernel,
        grid=(B // GB, B // GB),
        in_specs=[
            pl.BlockSpec((GB, D), lambda i, j: (i, 0)),
            pl.BlockSpec((GB, D), lambda i, j: (j, 0)),
        ],
        out_specs=pl.BlockSpec((1, 1), lambda i, j: (0, 0)),
        out_shape=jax.ShapeDtypeStruct((1, 1), jnp.float32),
    )(x, x)
    e_geom = geom_parts[0, 0] / (B * (B - 1))

    u = vals[:, 0]
    v = vals[:, 1]
    z = (W_comp_w[0, 0] * u + W_comp_w[0, 1] * v + W_comp_w[0, 2] * (u * v)
         + W_comp_b[0])
    e_comp = jax.nn.sigmoid(z)

    return e_splat + 0.01 * e_geom + 0.05 * e_comp


# speedup vs baseline: 32.1373x; 32.1373x over previous
import functools

import jax
import jax.numpy as jnp
from jax.experimental import pallas as pl
from jax.experimental.pallas import tpu as pltpu

KNN_K = 32
TEMP = 0.1
NEG_INIT = -3.0
NEG_DEAD = -4.0


def _fused_kernel(n_valid, cn, num_chunks,
                  x_ref, mu_ref, alpha_ref, w_ref, b_ref,
                  out_ref, s_ref, v_ref, i_ref, a_ref):
    j = pl.program_id(0)
    rb = x_ref.shape[0]

    @pl.when(j == 0)
    def _():
        v_ref[...] = jnp.full_like(v_ref, NEG_INIT)
        i_ref[...] = (2**30
                      + jax.lax.broadcasted_iota(jnp.int32, i_ref.shape, 1))
        a_ref[...] = jnp.zeros_like(a_ref)

    sims = jax.lax.dot_general(
        x_ref[...], mu_ref[...],
        (((1,), (1,)), ((), ())),
        preferred_element_type=jnp.float32,
    )
    gcol = jax.lax.broadcasted_iota(jnp.int32, (rb, cn), 1) + j * cn
    s_ref[...] = jnp.where(gcol < n_valid, sims, NEG_DEAD)
    alpha_b = alpha_ref[...][None, :]

    def step(_):
        v = v_ref[...]
        idx = i_ref[...]
        wv = jnp.min(v, axis=1, keepdims=True)
        wi = jnp.max(jnp.where(v == wv, idx, -1), axis=1, keepdims=True)
        s = s_ref[...]
        m = jnp.max(s, axis=1, keepdims=True)
        ci = jnp.min(jnp.where(s == m, gcol, 2**30), axis=1, keepdims=True)
        beats = (m > wv) | ((m == wv) & (ci < wi))
        colmask = gcol == ci
        s_ref[...] = jnp.where(colmask & beats, NEG_DEAD, s)
        asel = jnp.max(jnp.where(colmask, alpha_b, -1.0), axis=1, keepdims=True)
        upd = (v == wv) & (idx == wi) & beats
        v_ref[...] = jnp.where(upd, m, v)
        i_ref[...] = jnp.where(upd, ci, idx)
        a_ref[...] = jnp.where(upd, asel, a_ref[...])
        return jnp.any(beats)

    jax.lax.while_loop(lambda go: go, step, step(True))

    @pl.when(j == num_chunks - 1)
    def _():
        v = v_ref[...]
        a = a_ref[...]
        idx = i_ref[...]
        exponent = a * (v - 1.0) / TEMP
        emax = jnp.max(exponent, axis=1, keepdims=True)
        lse = jnp.log(jnp.sum(jnp.exp(exponent - emax), axis=1, keepdims=True)) + emax
        e_splat = -lse

        m1 = jnp.max(v, axis=1, keepdims=True)
        i1 = jnp.min(jnp.where(v == m1, idx, 2**30), axis=1, keepdims=True)
        m2 = jnp.max(jnp.where((v == m1) & (idx == i1), NEG_DEAD, v),
                     axis=1, keepdims=True)
        w0 = w_ref[0, 0]
        w1 = w_ref[0, 1]
        w2 = w_ref[0, 2]
        z = w0 * m1 + w1 * m2 + w2 * (m1 * m2) + b_ref[0]
        e_comp = jax.nn.sigmoid(z)
        out_ref[...] = e_splat + 0.05 * e_comp


def _geom_kernel(x_ref, xt_ref, o_ref):
    i = pl.program_id(0)
    j = pl.program_id(1)
    s = jax.lax.dot_general(
        x_ref[...], xt_ref[...],
        (((1,), (1,)), ((), ())),
        preferred_element_type=jnp.float32,
    )
    rb, cb = s.shape
    row = jax.lax.broadcasted_iota(jnp.int32, s.shape, 0) + i * rb
    col = jax.lax.broadcasted_iota(jnp.int32, s.shape, 1) + j * cb
    vals = -jnp.log(1.0 - s + 0.0001)
    vals = jnp.where(row == col, 0.0, vals)

    @pl.when((i == 0) & (j == 0))
    def _():
        o_ref[...] = jnp.zeros_like(o_ref)

    o_ref[...] += jnp.sum(vals)[None, None]


def _build_fused(B, D, N, CN, interpret=False):
    NP = ((N + CN - 1) // CN) * CN
    num_chunks = NP // CN
    return pl.pallas_call(
        functools.partial(_fused_kernel, N, CN, num_chunks),
        grid=(num_chunks,),
        in_specs=[
            pl.BlockSpec((B, D), lambda j: (0, 0)),
            pl.BlockSpec((CN, D), lambda j: (j, 0)),
            pl.BlockSpec((CN,), lambda j: (j,)),
            pl.BlockSpec(memory_space=pltpu.SMEM),
            pl.BlockSpec(memory_space=pltpu.SMEM),
        ],
        out_specs=pl.BlockSpec((B, 1), lambda j: (0, 0)),
        out_shape=jax.ShapeDtypeStruct((B, 1), jnp.float32),
        scratch_shapes=[
            pltpu.VMEM((B, CN), jnp.float32),
            pltpu.VMEM((B, KNN_K), jnp.float32),
            pltpu.VMEM((B, KNN_K), jnp.int32),
            pltpu.VMEM((B, KNN_K), jnp.float32),
        ],
        interpret=interpret,
    )


def kernel(x, mu, alpha, W_comp_w, W_comp_b, *, interpret=False, CN=512):
    B, D = x.shape
    N = mu.shape[0]
    NP = ((N + CN - 1) // CN) * CN
    mu_p = jnp.pad(mu, ((0, NP - N), (0, 0)))
    alpha_p = jnp.pad(alpha, (0, NP - N))

    e_main = _build_fused(B, D, N, CN, interpret=interpret)(
        x, mu_p, alpha_p, W_comp_w, W_comp_b)[:, 0]

    GB = min(1024, B)
    geom_parts = pl.pallas_call(
        _geom_kernel,
        grid=(B // GB, B // GB),
        in_specs=[
            pl.BlockSpec((GB, D), lambda i, j: (i, 0)),
            pl.BlockSpec((GB, D), lambda i, j: (j, 0)),
        ],
        out_specs=pl.BlockSpec((1, 1), lambda i, j: (0, 0)),
        out_shape=jax.ShapeDtypeStruct((1, 1), jnp.float32),
        interpret=interpret,
    )(x, x)
    e_geom = geom_parts[0, 0] / (B * (B - 1))

    return e_main + 0.01 * e_geom


# speedup vs baseline: 36.0393x; 1.1214x over previous
import functools

import jax
import jax.numpy as jnp
from jax.experimental import pallas as pl
from jax.experimental.pallas import tpu as pltpu

KNN_K = 32
TEMP = 0.1
NEG_INIT = -3.0
NEG_DEAD = -4.0


def _fused_kernel(n_valid, cn, num_chunks,
                  x_ref, mu_ref, alpha_ref, w_ref, b_ref,
                  out_ref, s_ref, v_ref, i_ref, a_ref):
    j = pl.program_id(0)
    rb = x_ref.shape[0]

    @pl.when(j == 0)
    def _():
        v_ref[...] = jnp.full_like(v_ref, NEG_INIT)
        i_ref[...] = (2**30
                      + jax.lax.broadcasted_iota(jnp.int32, i_ref.shape, 1))
        a_ref[...] = jnp.zeros_like(a_ref)

    sims = jax.lax.dot_general(
        x_ref[...], mu_ref[...],
        (((1,), (1,)), ((), ())),
        preferred_element_type=jnp.float32,
    )
    gcol = jax.lax.broadcasted_iota(jnp.int32, (rb, cn), 1) + j * cn
    s_ref[...] = jnp.where(gcol < n_valid, sims, NEG_DEAD)
    alpha_b = alpha_ref[...][None, :]

    def step(_):
        v = v_ref[...]
        idx = i_ref[...]
        wv = jnp.min(v, axis=1, keepdims=True)
        wi = jnp.max(jnp.where(v == wv, idx, -1), axis=1, keepdims=True)
        s = s_ref[...]
        m = jnp.max(s, axis=1, keepdims=True)
        ci = jnp.min(jnp.where(s == m, gcol, 2**30), axis=1, keepdims=True)
        beats = (m > wv) | ((m == wv) & (ci < wi))
        colmask = gcol == ci
        s_ref[...] = jnp.where(colmask & beats, NEG_DEAD, s)
        asel = jnp.max(jnp.where(colmask, alpha_b, -1.0), axis=1, keepdims=True)
        upd = (v == wv) & (idx == wi) & beats
        v_ref[...] = jnp.where(upd, m, v)
        i_ref[...] = jnp.where(upd, ci, idx)
        a_ref[...] = jnp.where(upd, asel, a_ref[...])
        return jnp.any(beats)

    jax.lax.while_loop(lambda go: go, step, step(True))

    @pl.when(j == num_chunks - 1)
    def _():
        v = v_ref[...]
        a = a_ref[...]
        idx = i_ref[...]
        exponent = a * (v - 1.0) / TEMP
        emax = jnp.max(exponent, axis=1, keepdims=True)
        lse = jnp.log(jnp.sum(jnp.exp(exponent - emax), axis=1, keepdims=True)) + emax
        e_splat = -lse

        m1 = jnp.max(v, axis=1, keepdims=True)
        i1 = jnp.min(jnp.where(v == m1, idx, 2**30), axis=1, keepdims=True)
        m2 = jnp.max(jnp.where((v == m1) & (idx == i1), NEG_DEAD, v),
                     axis=1, keepdims=True)
        w0 = w_ref[0, 0]
        w1 = w_ref[0, 1]
        w2 = w_ref[0, 2]
        z = w0 * m1 + w1 * m2 + w2 * (m1 * m2) + b_ref[0]
        e_comp = jax.nn.sigmoid(z)
        out_ref[...] = e_splat + 0.05 * e_comp


def _geom_kernel(x_ref, xt_ref, o_ref):
    i = pl.program_id(0)
    j = pl.program_id(1)
    s = jax.lax.dot_general(
        x_ref[...], xt_ref[...],
        (((1,), (1,)), ((), ())),
        preferred_element_type=jnp.float32,
    )
    rb, cb = s.shape
    row = jax.lax.broadcasted_iota(jnp.int32, s.shape, 0) + i * rb
    col = jax.lax.broadcasted_iota(jnp.int32, s.shape, 1) + j * cb
    vals = -jnp.log(1.0 - s + 0.0001)
    vals = jnp.where(row == col, 0.0, vals)

    @pl.when((i == 0) & (j == 0))
    def _():
        o_ref[...] = jnp.zeros_like(o_ref)

    o_ref[...] += jnp.sum(vals)[None, None]


def _build_fused(B, D, N, CN, interpret=False):
    NP = ((N + CN - 1) // CN) * CN
    num_chunks = NP // CN
    return pl.pallas_call(
        functools.partial(_fused_kernel, N, CN, num_chunks),
        grid=(num_chunks,),
        in_specs=[
            pl.BlockSpec((B, D), lambda j: (0, 0)),
            pl.BlockSpec((CN, D), lambda j: (j, 0)),
            pl.BlockSpec((CN,), lambda j: (j,)),
            pl.BlockSpec(memory_space=pltpu.SMEM),
            pl.BlockSpec(memory_space=pltpu.SMEM),
        ],
        out_specs=pl.BlockSpec((B, 1), lambda j: (0, 0)),
        out_shape=jax.ShapeDtypeStruct((B, 1), jnp.float32),
        scratch_shapes=[
            pltpu.VMEM((B, CN), jnp.float32),
            pltpu.VMEM((B, KNN_K), jnp.float32),
            pltpu.VMEM((B, KNN_K), jnp.int32),
            pltpu.VMEM((B, KNN_K), jnp.float32),
        ],
        interpret=interpret,
    )


def kernel(x, mu, alpha, W_comp_w, W_comp_b, *, interpret=False, CN=1024):
    B, D = x.shape
    N = mu.shape[0]
    NP = ((N + CN - 1) // CN) * CN
    mu_p = jnp.pad(mu, ((0, NP - N), (0, 0)))
    alpha_p = jnp.pad(alpha, (0, NP - N))

    e_main = _build_fused(B, D, N, CN, interpret=interpret)(
        x, mu_p, alpha_p, W_comp_w, W_comp_b)[:, 0]

    GB = min(1024, B)
    geom_parts = pl.pallas_call(
        _geom_kernel,
        grid=(B // GB, B // GB),
        in_specs=[
            pl.BlockSpec((GB, D), lambda i, j: (i, 0)),
            pl.BlockSpec((GB, D), lambda i, j: (j, 0)),
        ],
        out_specs=pl.BlockSpec((1, 1), lambda i, j: (0, 0)),
        out_shape=jax.ShapeDtypeStruct((1, 1), jnp.float32),
        interpret=interpret,
    )(x, x)
    e_geom = geom_parts[0, 0] / (B * (B - 1))

    return e_main + 0.01 * e_geom
